# baseline (speedup 1.0000x reference)
import functools

import jax
import jax.numpy as jnp
from jax import lax
from jax.experimental import pallas as pl
from jax.experimental.pallas import tpu as pltpu
from jax.experimental.pallas import tpu_sc as plsc

N = 10000
E = 320000
D_IN = 128
HID = 16
HEADS = 8
D_OUT = 128
ROW = 144

NCORE = 2
NSUB = 16
NW = NCORE * NSUB
EPW = E // NW
B = 80
CHUNKS = EPW // B
RPS = N // NSUB
BLK = 1000
GRID = N // BLK


def _k1_body(x_ref, w_ref, asrc_ref, adst_ref, tab_ref, ad_ref):
    h = jnp.dot(x_ref[...], w_ref[...], preferred_element_type=jnp.float32)
    a_s = jnp.dot(h, asrc_ref[...], preferred_element_type=jnp.float32)
    a_d = jnp.dot(h, adst_ref[...], preferred_element_type=jnp.float32)
    tab_ref[...] = jnp.concatenate(
        [h, a_s, jnp.zeros((h.shape[0], 8), jnp.float32)], axis=1)
    ad_ref[...] = a_d


def _k3_body(p_ref, tab_ref, ad_ref, b1_ref, w2_ref, a2_ref, e8_ref,
             tab2_ref, a2out_ref):
    p = p_ref[...]
    h1 = tab_ref[:, 0:128]
    a_s = tab_ref[:, 128:136]
    e = a_s + ad_ref[...]
    w = jnp.exp(jnp.where(e >= 0, e, 0.2 * e))
    wexp = jnp.dot(w, e8_ref[...], preferred_element_type=jnp.float32)
    num = p[0, :, 0:128] + p[1, :, 0:128] + h1 * wexp
    den = p[0, :, 128:136] + p[1, :, 128:136] + w
    den_exp = jnp.dot(den, e8_ref[...], preferred_element_type=jnp.float32)
    hmid = jnp.maximum(num / (den_exp + 1e-16) + b1_ref[...], 0.0)
    h2 = jnp.dot(hmid, w2_ref[...], preferred_element_type=jnp.float32)
    a2 = jnp.dot(h2, a2_ref[...], preferred_element_type=jnp.float32)
    tab2_ref[...] = jnp.concatenate(
        [h2, a2, jnp.zeros((h2.shape[0], 8), jnp.float32)], axis=1)
    a2out_ref[...] = a2


def _k5_body(p_ref, tab2_ref, a2_ref, b2_ref, out_ref):
    p = p_ref[...]
    h2 = tab2_ref[:, 0:128]
    a2 = a2_ref[...]
    e = a2[:, 0:1] + a2[:, 1:2]
    w = jnp.exp(jnp.where(e >= 0, e, 0.2 * e))
    num = p[0, :, 0:128] + p[1, :, 0:128] + h2 * w
    den = p[0, :, 128:129] + p[1, :, 128:129] + w
    out_ref[...] = num / (den + 1e-16) + b2_ref[...]


NBLK = N // B


def _zero_and_load(gbuf, acc, s, sem):
    zero = jnp.zeros((16,), jnp.float32)

    def zb(i, carry):
        row = i // (ROW // 16)
        colc = i % (ROW // 16)
        gbuf[row, pl.ds(colc * 16, 16)] = zero
        return carry

    lax.fori_loop(0, B * (ROW // 16), zb, 0)
    for t in range((NBLK + NSUB - 1) // NSUB):
        b = s + t * NSUB

        @pl.when(b < NBLK)
        def _():
            pltpu.async_copy(gbuf, acc.at[pl.ds(b * B, B)], sem)

    for t in range((NBLK + NSUB - 1) // NSUB):
        b = s + t * NSUB

        @pl.when(b < NBLK)
        def _():
            pltpu.make_async_copy(gbuf, acc.at[pl.ds(b * B, B)], sem).wait()


def _write_out(acc, out_hbm, c, s, sem):
    for t in range((NBLK + NSUB - 1) // NSUB):
        b = s + t * NSUB

        @pl.when(b < NBLK)
        def _():
            pltpu.async_copy(acc.at[pl.ds(b * B, B)],
                             out_hbm.at[c, pl.ds(b * B, B)], sem)

    for t in range((NBLK + NSUB - 1) // NSUB):
        b = s + t * NSUB

        @pl.when(b < NBLK)
        def _():
            pltpu.make_async_copy(acc.at[pl.ds(b * B, B)],
                                  out_hbm.at[c, pl.ds(b * B, B)], sem).wait()


def _compute_l1(gbuf, adbuf, iota):
    @plsc.parallel_loop(0, B // 2, 1, unroll=4)
    def _pair(j2):
        j = j2 * 2
        rows = j + (iota >> 3)
        cols = 128 + (iota & 7)
        a_s = plsc.load_gather(gbuf, [rows, cols])
        a_d = plsc.load_gather(adbuf, [rows, iota & 7])
        e = a_s + a_d
        e = jnp.where(e >= 0, e, 0.2 * e)
        w = jnp.exp(e)
        for jj in range(2):
            for k in range(HEADS):
                ws = jnp.take_along_axis(
                    w, jnp.full((16,), jj * 8 + k, jnp.int32), axis=0)
                hv = gbuf[j + jj, pl.ds(k * 16, 16)]
                gbuf[j + jj, pl.ds(k * 16, 16)] = hv * ws
        plsc.store_scatter(gbuf, [rows, cols], w)


def _compute_l2(gbuf, adbuf, iota):
    @plsc.parallel_loop(0, B // 16, 1)
    def _group(g):
        rows = g * 16 + iota
        a_s = plsc.load_gather(gbuf, [rows, jnp.broadcast_to(128, (16,))])
        a_d = plsc.load_gather(adbuf, [rows, jnp.broadcast_to(1, (16,))])
        e = a_s + a_d
        e = jnp.where(e >= 0, e, 0.2 * e)
        w16 = jnp.exp(e)
        for i in range(16):
            j = g * 16 + i
            ws = jnp.take_along_axis(
                w16, jnp.full((16,), i, jnp.int32), axis=0)
            for k in range(8):
                hv = gbuf[j, pl.ds(k * 16, 16)]
                gbuf[j, pl.ds(k * 16, 16)] = hv * ws
            gbuf[j, pl.ds(128, 16)] = jnp.where(iota == 0, ws, 0.0)


def _make_sc_body(compute):
    def body(tab_hbm, adst_hbm, idx_hbm, out_hbm,
             acc, idxb, adb, gb, dstv,
             sg0, sg1, sg2, sa0, sa1, sa2, ss0, ss1, ss2):
        sems_g = (sg0, sg1, sg2)
        sems_a = (sa0, sa1, sa2)
        sems_s = (ss0, ss1, ss2)
        c = lax.axis_index("c")
        s = lax.axis_index("s")
        wid = c * NSUB + s
        iota = lax.iota(jnp.int32, 16)
        rbase = wid * CHUNKS

        def issue(t, b):
            pltpu.sync_copy(idx_hbm.at[rbase + t], idxb.at[b])
            pltpu.async_copy(adst_hbm.at[idxb.at[b, 1]], adb.at[b], sems_a[b])
            pltpu.async_copy(tab_hbm.at[idxb.at[b, 0]], gb.at[b], sems_g[b])

        def wait_gathers(b):
            pltpu.make_async_copy(
                adst_hbm.at[idxb.at[b, 1]], adb.at[b], sems_a[b]).wait()
            pltpu.make_async_copy(
                tab_hbm.at[idxb.at[b, 0]], gb.at[b], sems_g[b]).wait()

        def wait_scat(b):
            pltpu.make_async_copy(
                gb.at[b], acc.at[dstv.at[b]], sems_s[b]).wait()

        def compute_chunk(b):
            for i in range(B // 16):
                dstv[b, pl.ds(i * 16, 16)] = idxb[b, 1, pl.ds(i * 16, 16)]
            compute(gb.at[b], adb.at[b], iota)
            pltpu.async_copy(gb.at[b], acc.at[dstv.at[b]], sems_s[b],
                             add=True)

        _zero_and_load(gb.at[0], acc, s, sg0)
        plsc.subcore_barrier()

        issue(0, 0)
        issue(1, 1)

        def step(k, carry):
            for b in range(3):
                t = k * 3 + b

                @pl.when(t >= 1)
                def _():
                    wait_scat((b + 2) % 3)

                issue(t + 2, (b + 2) % 3)
                wait_gathers(b)
                compute_chunk(b)
            return carry

        lax.fori_loop(0, (CHUNKS - 2) // 3, step, 0)
        wait_scat(2)
        wait_gathers(0)
        compute_chunk(0)
        wait_gathers(1)
        compute_chunk(1)
        wait_scat(0)
        wait_scat(1)
        plsc.subcore_barrier()
        _write_out(acc, out_hbm, c, s, sg0)

    return body


def _make_sc_call(compute):
    mesh = plsc.VectorSubcoreMesh(core_axis_name="c", subcore_axis_name="s",
                                  num_cores=NCORE, num_subcores=NSUB)
    return pl.kernel(
        _make_sc_body(compute),
        out_type=jax.ShapeDtypeStruct((NCORE, N, ROW), jnp.float32),
        mesh=mesh,
        compiler_params=pltpu.CompilerParams(use_tc_tiling_on_sc=False,
                                             needs_layout_passes=False),
        scratch_types=[
            pltpu.VMEM_SHARED((N, ROW), jnp.float32),
            pltpu.VMEM((3, 2, B), jnp.int32),
            pltpu.VMEM((3, B, 8), jnp.float32),
            pltpu.VMEM((3, B, ROW), jnp.float32),
            pltpu.VMEM((3, B), jnp.int32),
        ] + [pltpu.SemaphoreType.DMA] * 9,
    )


@jax.jit
def kernel(x, edge_index, W1, att_src1, att_dst1, b1, W2, att_src2,
           att_dst2, b2):
    f32 = jnp.float32
    idxpk = jnp.stack([edge_index[0].reshape(E // B, B),
                       edge_index[1].reshape(E // B, B)], axis=1)

    eye8 = jnp.eye(HEADS, dtype=f32)
    asrc1_m = (att_src1[:, :, None] * eye8[:, None, :]).reshape(HEADS * HID, HEADS)
    adst1_m = (att_dst1[:, :, None] * eye8[:, None, :]).reshape(HEADS * HID, HEADS)
    e8 = jnp.repeat(eye8, HID, axis=1)
    a2_m = jnp.concatenate(
        [att_src2.reshape(D_OUT, 1), att_dst2.reshape(D_OUT, 1),
         jnp.zeros((D_OUT, 6), f32)], axis=1)

    k1 = pl.pallas_call(
        _k1_body,
        grid=(GRID,),
        in_specs=[
            pl.BlockSpec((BLK, D_IN), lambda i: (i, 0)),
            pl.BlockSpec((D_IN, HEADS * HID), lambda i: (0, 0)),
            pl.BlockSpec((HEADS * HID, HEADS), lambda i: (0, 0)),
            pl.BlockSpec((HEADS * HID, HEADS), lambda i: (0, 0)),
        ],
        out_specs=[
            pl.BlockSpec((BLK, ROW), lambda i: (i, 0)),
            pl.BlockSpec((BLK, HEADS), lambda i: (i, 0)),
        ],
        out_shape=[
            jax.ShapeDtypeStruct((N, ROW), f32),
            jax.ShapeDtypeStruct((N, HEADS), f32),
        ],
    )
    tab1, adst1 = k1(x, W1, asrc1_m, adst1_m)

    sc_l1 = _make_sc_call(_compute_l1)
    p1 = sc_l1(tab1, adst1, idxpk)

    k3 = pl.pallas_call(
        _k3_body,
        grid=(GRID,),
        in_specs=[
            pl.BlockSpec((NCORE, BLK, ROW), lambda i: (0, i, 0)),
            pl.BlockSpec((BLK, ROW), lambda i: (i, 0)),
            pl.BlockSpec((BLK, HEADS), lambda i: (i, 0)),
            pl.BlockSpec((1, HEADS * HID), lambda i: (0, 0)),
            pl.BlockSpec((HEADS * HID, D_OUT), lambda i: (0, 0)),
            pl.BlockSpec((D_OUT, HEADS), lambda i: (0, 0)),
            pl.BlockSpec((HEADS, HEADS * HID), lambda i: (0, 0)),
        ],
        out_specs=[
            pl.BlockSpec((BLK, ROW), lambda i: (i, 0)),
            pl.BlockSpec((BLK, HEADS), lambda i: (i, 0)),
        ],
        out_shape=[
            jax.ShapeDtypeStruct((N, ROW), f32),
            jax.ShapeDtypeStruct((N, HEADS), f32),
        ],
    )
    tab2, a2all = k3(p1, tab1, adst1, b1.reshape(1, -1), W2, a2_m, e8)

    sc_l2 = _make_sc_call(_compute_l2)
    p2 = sc_l2(tab2, a2all, idxpk)

    k5 = pl.pallas_call(
        _k5_body,
        grid=(GRID,),
        in_specs=[
            pl.BlockSpec((NCORE, BLK, ROW), lambda i: (0, i, 0)),
            pl.BlockSpec((BLK, ROW), lambda i: (i, 0)),
            pl.BlockSpec((BLK, HEADS), lambda i: (i, 0)),
            pl.BlockSpec((1, D_OUT), lambda i: (0, 0)),
        ],
        out_specs=pl.BlockSpec((BLK, D_OUT), lambda i: (i, 0)),
        out_shape=jax.ShapeDtypeStruct((N, D_OUT), f32),
    )
    return k5(p2, tab2, a2all, b2.reshape(1, -1))


# speedup vs baseline: 1.0429x; 1.0429x over previous
import functools

import jax
import jax.numpy as jnp
from jax import lax
from jax.experimental import pallas as pl
from jax.experimental.pallas import tpu as pltpu
from jax.experimental.pallas import tpu_sc as plsc

N = 10000
E = 320000
D_IN = 128
HID = 16
HEADS = 8
D_OUT = 128
ROW = 144

NCORE = 2
NSUB = 16
NW = NCORE * NSUB
EPW = E // NW
B = 80
CHUNKS = EPW // B
RPS = N // NSUB
BLK = 1000
GRID = N // BLK


def _k1_body(x_ref, w_ref, asrc_ref, adst_ref, tab_ref, ad_ref):
    h = jnp.dot(x_ref[...], w_ref[...], preferred_element_type=jnp.float32)
    a_s = jnp.dot(h, asrc_ref[...], preferred_element_type=jnp.float32)
    a_d = jnp.dot(h, adst_ref[...], preferred_element_type=jnp.float32)
    tab_ref[...] = jnp.concatenate(
        [h, a_s, jnp.zeros((h.shape[0], 8), jnp.float32)], axis=1)
    ad_ref[...] = a_d


def _k3_body(p_ref, tab_ref, ad_ref, b1_ref, w2_ref, a2_ref, e8_ref,
             tab2_ref, a2out_ref):
    p = p_ref[...]
    h1 = tab_ref[:, 0:128]
    a_s = tab_ref[:, 128:136]
    e = a_s + ad_ref[...]
    w = jnp.exp(jnp.where(e >= 0, e, 0.2 * e))
    wexp = jnp.dot(w, e8_ref[...], preferred_element_type=jnp.float32)
    num = p[0, :, 0:128] + p[1, :, 0:128] + h1 * wexp
    den = p[0, :, 128:136] + p[1, :, 128:136] + w
    den_exp = jnp.dot(den, e8_ref[...], preferred_element_type=jnp.float32)
    hmid = jnp.maximum(num / (den_exp + 1e-16) + b1_ref[...], 0.0)
    h2 = jnp.dot(hmid, w2_ref[...], preferred_element_type=jnp.float32)
    a2 = jnp.dot(h2, a2_ref[...], preferred_element_type=jnp.float32)
    tab2_ref[...] = jnp.concatenate(
        [h2, a2, jnp.zeros((h2.shape[0], 8), jnp.float32)], axis=1)
    a2out_ref[...] = a2


def _k5_body(p_ref, tab2_ref, a2_ref, b2_ref, out_ref):
    p = p_ref[...]
    h2 = tab2_ref[:, 0:128]
    a2 = a2_ref[...]
    e = a2[:, 0:1] + a2[:, 1:2]
    w = jnp.exp(jnp.where(e >= 0, e, 0.2 * e))
    num = p[0, :, 0:128] + p[1, :, 0:128] + h2 * w
    den = p[0, :, 128:129] + p[1, :, 128:129] + w
    out_ref[...] = num / (den + 1e-16) + b2_ref[...]


NBLK = N // B


def _zero_and_load(gbuf, acc, s, sem):
    zero = jnp.zeros((16,), jnp.float32)

    def zb(i, carry):
        row = i // (ROW // 16)
        colc = i % (ROW // 16)
        gbuf[row, pl.ds(colc * 16, 16)] = zero
        return carry

    lax.fori_loop(0, B * (ROW // 16), zb, 0)
    for t in range((NBLK + NSUB - 1) // NSUB):
        b = s + t * NSUB

        @pl.when(b < NBLK)
        def _():
            pltpu.async_copy(gbuf, acc.at[pl.ds(b * B, B)], sem)

    for t in range((NBLK + NSUB - 1) // NSUB):
        b = s + t * NSUB

        @pl.when(b < NBLK)
        def _():
            pltpu.make_async_copy(gbuf, acc.at[pl.ds(b * B, B)], sem).wait()


def _write_out(acc, out_hbm, c, s, sem):
    for t in range((NBLK + NSUB - 1) // NSUB):
        b = s + t * NSUB

        @pl.when(b < NBLK)
        def _():
            pltpu.async_copy(acc.at[pl.ds(b * B, B)],
                             out_hbm.at[c, pl.ds(b * B, B)], sem)

    for t in range((NBLK + NSUB - 1) // NSUB):
        b = s + t * NSUB

        @pl.when(b < NBLK)
        def _():
            pltpu.make_async_copy(acc.at[pl.ds(b * B, B)],
                                  out_hbm.at[c, pl.ds(b * B, B)], sem).wait()


def _compute_l1(gbuf, adbuf, iota):
    @plsc.parallel_loop(0, B // 2, 1, unroll=2)
    def _pair(j2):
        j = j2 * 2
        rows = j + (iota >> 3)
        cols = 128 + (iota & 7)
        a_s = plsc.load_gather(gbuf, [rows, cols])
        a_d = plsc.load_gather(adbuf, [rows, iota & 7])
        e = a_s + a_d
        e = jnp.where(e >= 0, e, 0.2 * e)
        w = jnp.exp(e)
        for jj in range(2):
            for k in range(HEADS):
                ws = jnp.take_along_axis(
                    w, jnp.full((16,), jj * 8 + k, jnp.int32), axis=0)
                hv = gbuf[j + jj, pl.ds(k * 16, 16)]
                gbuf[j + jj, pl.ds(k * 16, 16)] = hv * ws
        plsc.store_scatter(gbuf, [rows, cols], w)


def _compute_l2(gbuf, adbuf, iota):
    @plsc.parallel_loop(0, B // 16, 1)
    def _group(g):
        rows = g * 16 + iota
        a_s = plsc.load_gather(gbuf, [rows, jnp.broadcast_to(128, (16,))])
        a_d = plsc.load_gather(adbuf, [rows, jnp.broadcast_to(1, (16,))])
        e = a_s + a_d
        e = jnp.where(e >= 0, e, 0.2 * e)
        w16 = jnp.exp(e)
        for i in range(16):
            j = g * 16 + i
            ws = jnp.take_along_axis(
                w16, jnp.full((16,), i, jnp.int32), axis=0)
            for k in range(8):
                hv = gbuf[j, pl.ds(k * 16, 16)]
                gbuf[j, pl.ds(k * 16, 16)] = hv * ws
            gbuf[j, pl.ds(128, 16)] = jnp.where(iota == 0, ws, 0.0)


def _make_sc_body(compute):
    def body(tab_hbm, adst_hbm, idx_hbm, out_hbm,
             acc, idxb, adb, gb, dstv,
             sg0, sg1, sg2, sa0, sa1, sa2, ss0, ss1, ss2):
        sems_g = (sg0, sg1, sg2)
        sems_a = (sa0, sa1, sa2)
        sems_s = (ss0, ss1, ss2)
        c = lax.axis_index("c")
        s = lax.axis_index("s")
        wid = c * NSUB + s
        iota = lax.iota(jnp.int32, 16)
        rbase = wid * CHUNKS

        def issue(t, b):
            pltpu.sync_copy(idx_hbm.at[rbase + t], idxb.at[b])
            pltpu.async_copy(adst_hbm.at[idxb.at[b, 1]], adb.at[b], sems_a[b])
            pltpu.async_copy(tab_hbm.at[idxb.at[b, 0]], gb.at[b], sems_g[b])

        def wait_gathers(b):
            pltpu.make_async_copy(
                adst_hbm.at[idxb.at[b, 1]], adb.at[b], sems_a[b]).wait()
            pltpu.make_async_copy(
                tab_hbm.at[idxb.at[b, 0]], gb.at[b], sems_g[b]).wait()

        def wait_scat(b):
            pltpu.make_async_copy(
                gb.at[b], acc.at[dstv.at[b]], sems_s[b]).wait()

        def compute_chunk(b):
            for i in range(B // 16):
                dstv[b, pl.ds(i * 16, 16)] = idxb[b, 1, pl.ds(i * 16, 16)]
            compute(gb.at[b], adb.at[b], iota)
            pltpu.async_copy(gb.at[b], acc.at[dstv.at[b]], sems_s[b],
                             add=True)

        _zero_and_load(gb.at[0], acc, s, sg0)
        plsc.subcore_barrier()

        issue(0, 0)
        issue(1, 1)

        def step(k, carry):
            for b in range(3):
                t = k * 3 + b

                @pl.when(t >= 1)
                def _():
                    wait_scat((b + 2) % 3)

                issue(t + 2, (b + 2) % 3)
                wait_gathers(b)
                compute_chunk(b)
            return carry

        lax.fori_loop(0, (CHUNKS - 2) // 3, step, 0)
        wait_scat(2)
        wait_gathers(0)
        compute_chunk(0)
        wait_gathers(1)
        compute_chunk(1)
        wait_scat(0)
        wait_scat(1)
        plsc.subcore_barrier()
        _write_out(acc, out_hbm, c, s, sg0)

    return body


def _make_sc_call(compute):
    mesh = plsc.VectorSubcoreMesh(core_axis_name="c", subcore_axis_name="s",
                                  num_cores=NCORE, num_subcores=NSUB)
    return pl.kernel(
        _make_sc_body(compute),
        out_type=jax.ShapeDtypeStruct((NCORE, N, ROW), jnp.float32),
        mesh=mesh,
        compiler_params=pltpu.CompilerParams(use_tc_tiling_on_sc=False,
                                             needs_layout_passes=False),
        scratch_types=[
            pltpu.VMEM_SHARED((N, ROW), jnp.float32),
            pltpu.VMEM((3, 2, B), jnp.int32),
            pltpu.VMEM((3, B, 8), jnp.float32),
            pltpu.VMEM((3, B, ROW), jnp.float32),
            pltpu.VMEM((3, B), jnp.int32),
        ] + [pltpu.SemaphoreType.DMA] * 9,
    )


@jax.jit
def kernel(x, edge_index, W1, att_src1, att_dst1, b1, W2, att_src2,
           att_dst2, b2):
    f32 = jnp.float32
    idxpk = jnp.stack([edge_index[0].reshape(E // B, B),
                       edge_index[1].reshape(E // B, B)], axis=1)

    eye8 = jnp.eye(HEADS, dtype=f32)
    asrc1_m = (att_src1[:, :, None] * eye8[:, None, :]).reshape(HEADS * HID, HEADS)
    adst1_m = (att_dst1[:, :, None] * eye8[:, None, :]).reshape(HEADS * HID, HEADS)
    e8 = jnp.repeat(eye8, HID, axis=1)
    a2_m = jnp.concatenate(
        [att_src2.reshape(D_OUT, 1), att_dst2.reshape(D_OUT, 1),
         jnp.zeros((D_OUT, 6), f32)], axis=1)

    k1 = pl.pallas_call(
        _k1_body,
        grid=(GRID,),
        in_specs=[
            pl.BlockSpec((BLK, D_IN), lambda i: (i, 0)),
            pl.BlockSpec((D_IN, HEADS * HID), lambda i: (0, 0)),
            pl.BlockSpec((HEADS * HID, HEADS), lambda i: (0, 0)),
            pl.BlockSpec((HEADS * HID, HEADS), lambda i: (0, 0)),
        ],
        out_specs=[
            pl.BlockSpec((BLK, ROW), lambda i: (i, 0)),
            pl.BlockSpec((BLK, HEADS), lambda i: (i, 0)),
        ],
        out_shape=[
            jax.ShapeDtypeStruct((N, ROW), f32),
            jax.ShapeDtypeStruct((N, HEADS), f32),
        ],
    )
    tab1, adst1 = k1(x, W1, asrc1_m, adst1_m)

    sc_l1 = _make_sc_call(_compute_l1)
    p1 = sc_l1(tab1, adst1, idxpk)

    k3 = pl.pallas_call(
        _k3_body,
        grid=(GRID,),
        in_specs=[
            pl.BlockSpec((NCORE, BLK, ROW), lambda i: (0, i, 0)),
            pl.BlockSpec((BLK, ROW), lambda i: (i, 0)),
            pl.BlockSpec((BLK, HEADS), lambda i: (i, 0)),
            pl.BlockSpec((1, HEADS * HID), lambda i: (0, 0)),
            pl.BlockSpec((HEADS * HID, D_OUT), lambda i: (0, 0)),
            pl.BlockSpec((D_OUT, HEADS), lambda i: (0, 0)),
            pl.BlockSpec((HEADS, HEADS * HID), lambda i: (0, 0)),
        ],
        out_specs=[
            pl.BlockSpec((BLK, ROW), lambda i: (i, 0)),
            pl.BlockSpec((BLK, HEADS), lambda i: (i, 0)),
        ],
        out_shape=[
            jax.ShapeDtypeStruct((N, ROW), f32),
            jax.ShapeDtypeStruct((N, HEADS), f32),
        ],
    )
    tab2, a2all = k3(p1, tab1, adst1, b1.reshape(1, -1), W2, a2_m, e8)

    sc_l2 = _make_sc_call(_compute_l2)
    p2 = sc_l2(tab2, a2all, idxpk)

    k5 = pl.pallas_call(
        _k5_body,
        grid=(GRID,),
        in_specs=[
            pl.BlockSpec((NCORE, BLK, ROW), lambda i: (0, i, 0)),
            pl.BlockSpec((BLK, ROW), lambda i: (i, 0)),
            pl.BlockSpec((BLK, HEADS), lambda i: (i, 0)),
            pl.BlockSpec((1, D_OUT), lambda i: (0, 0)),
        ],
        out_specs=pl.BlockSpec((BLK, D_OUT), lambda i: (i, 0)),
        out_shape=jax.ShapeDtypeStruct((N, D_OUT), f32),
    )
    return k5(p2, tab2, a2all, b2.reshape(1, -1))


# speedup vs baseline: 1.2590x; 1.2073x over previous
import functools

import jax
import jax.numpy as jnp
from jax import lax
from jax.experimental import pallas as pl
from jax.experimental.pallas import tpu as pltpu
from jax.experimental.pallas import tpu_sc as plsc

N = 10000
E = 320000
D_IN = 128
HID = 16
HEADS = 8
D_OUT = 128
ROW = 144

NCORE = 2
NSUB = 16
NW = NCORE * NSUB
EPW = E // NW
B = 80
CHUNKS = EPW // B
RPS = N // NSUB
BLK = 1000
GRID = N // BLK


def _k1_body(x_ref, w_ref, asrc_ref, adst_ref, tab_ref, ad_ref):
    h = jnp.dot(x_ref[...], w_ref[...], preferred_element_type=jnp.float32)
    a_s = jnp.dot(h, asrc_ref[...], preferred_element_type=jnp.float32)
    a_d = jnp.dot(h, adst_ref[...], preferred_element_type=jnp.float32)
    tab_ref[...] = jnp.concatenate(
        [h, a_s, jnp.zeros((h.shape[0], 8), jnp.float32)], axis=1)
    ad_ref[...] = a_d


def _k3_body(p_ref, tab_ref, ad_ref, b1_ref, w2_ref, a2_ref, e8_ref,
             tab2_ref, a2out_ref):
    p = p_ref[...]
    h1 = tab_ref[:, 0:128]
    a_s = tab_ref[:, 128:136]
    e = a_s + ad_ref[...]
    w = jnp.exp(jnp.where(e >= 0, e, 0.2 * e))
    wexp = jnp.dot(w, e8_ref[...], preferred_element_type=jnp.float32)
    num = p[0, :, 0:128] + p[1, :, 0:128] + h1 * wexp
    den = p[0, :, 128:136] + p[1, :, 128:136] + w
    den_exp = jnp.dot(den, e8_ref[...], preferred_element_type=jnp.float32)
    hmid = jnp.maximum(num / (den_exp + 1e-16) + b1_ref[...], 0.0)
    h2 = jnp.dot(hmid, w2_ref[...], preferred_element_type=jnp.float32)
    a2 = jnp.dot(h2, a2_ref[...], preferred_element_type=jnp.float32)
    tab2_ref[...] = jnp.concatenate(
        [h2, a2, jnp.zeros((h2.shape[0], 8), jnp.float32)], axis=1)
    a2out_ref[...] = a2


def _k5_body(p_ref, tab2_ref, a2_ref, b2_ref, out_ref):
    p = p_ref[...]
    h2 = tab2_ref[:, 0:128]
    a2 = a2_ref[...]
    e = a2[:, 0:1] + a2[:, 1:2]
    w = jnp.exp(jnp.where(e >= 0, e, 0.2 * e))
    num = p[0, :, 0:128] + p[1, :, 0:128] + h2 * w
    den = p[0, :, 128:129] + p[1, :, 128:129] + w
    out_ref[...] = num / (den + 1e-16) + b2_ref[...]


NBLK = N // B


def _zero_and_load(gbuf, acc, s, sem):
    zero = jnp.zeros((16,), jnp.float32)

    def zb(i, carry):
        row = i // (ROW // 16)
        colc = i % (ROW // 16)
        gbuf[row, pl.ds(colc * 16, 16)] = zero
        return carry

    lax.fori_loop(0, B * (ROW // 16), zb, 0)
    for t in range((NBLK + NSUB - 1) // NSUB):
        b = s + t * NSUB

        @pl.when(b < NBLK)
        def _():
            pltpu.async_copy(gbuf, acc.at[pl.ds(b * B, B)], sem)

    for t in range((NBLK + NSUB - 1) // NSUB):
        b = s + t * NSUB

        @pl.when(b < NBLK)
        def _():
            pltpu.make_async_copy(gbuf, acc.at[pl.ds(b * B, B)], sem).wait()


def _write_out(acc, out_hbm, c, s, sem):
    for t in range((NBLK + NSUB - 1) // NSUB):
        b = s + t * NSUB

        @pl.when(b < NBLK)
        def _():
            pltpu.async_copy(acc.at[pl.ds(b * B, B)],
                             out_hbm.at[c, pl.ds(b * B, B)], sem)

    for t in range((NBLK + NSUB - 1) // NSUB):
        b = s + t * NSUB

        @pl.when(b < NBLK)
        def _():
            pltpu.make_async_copy(acc.at[pl.ds(b * B, B)],
                                  out_hbm.at[c, pl.ds(b * B, B)], sem).wait()


def _compute_l1(gbuf, adbuf, iota):
    return
    @plsc.parallel_loop(0, B // 2, 1, unroll=2)
    def _pair(j2):
        j = j2 * 2
        rows = j + (iota >> 3)
        cols = 128 + (iota & 7)
        a_s = plsc.load_gather(gbuf, [rows, cols])
        a_d = plsc.load_gather(adbuf, [rows, iota & 7])
        e = a_s + a_d
        e = jnp.where(e >= 0, e, 0.2 * e)
        w = jnp.exp(e)
        for jj in range(2):
            for k in range(HEADS):
                ws = jnp.take_along_axis(
                    w, jnp.full((16,), jj * 8 + k, jnp.int32), axis=0)
                hv = gbuf[j + jj, pl.ds(k * 16, 16)]
                gbuf[j + jj, pl.ds(k * 16, 16)] = hv * ws
        plsc.store_scatter(gbuf, [rows, cols], w)


def _compute_l2(gbuf, adbuf, iota):
    return
    @plsc.parallel_loop(0, B // 16, 1)
    def _group(g):
        rows = g * 16 + iota
        a_s = plsc.load_gather(gbuf, [rows, jnp.broadcast_to(128, (16,))])
        a_d = plsc.load_gather(adbuf, [rows, jnp.broadcast_to(1, (16,))])
        e = a_s + a_d
        e = jnp.where(e >= 0, e, 0.2 * e)
        w16 = jnp.exp(e)
        for i in range(16):
            j = g * 16 + i
            ws = jnp.take_along_axis(
                w16, jnp.full((16,), i, jnp.int32), axis=0)
            for k in range(8):
                hv = gbuf[j, pl.ds(k * 16, 16)]
                gbuf[j, pl.ds(k * 16, 16)] = hv * ws
            gbuf[j, pl.ds(128, 16)] = jnp.where(iota == 0, ws, 0.0)


def _make_sc_body(compute):
    def body(tab_hbm, adst_hbm, idx_hbm, out_hbm,
             acc, idxb, adb, gb, dstv,
             sg0, sg1, sg2, sa0, sa1, sa2, ss0, ss1, ss2):
        sems_g = (sg0, sg1, sg2)
        sems_a = (sa0, sa1, sa2)
        sems_s = (ss0, ss1, ss2)
        c = lax.axis_index("c")
        s = lax.axis_index("s")
        wid = c * NSUB + s
        iota = lax.iota(jnp.int32, 16)
        rbase = wid * CHUNKS

        def issue(t, b):
            pltpu.sync_copy(idx_hbm.at[rbase + t], idxb.at[b])
            pltpu.async_copy(adst_hbm.at[idxb.at[b, 1]], adb.at[b], sems_a[b])
            pltpu.async_copy(tab_hbm.at[idxb.at[b, 0]], gb.at[b], sems_g[b])

        def wait_gathers(b):
            pltpu.make_async_copy(
                adst_hbm.at[idxb.at[b, 1]], adb.at[b], sems_a[b]).wait()
            pltpu.make_async_copy(
                tab_hbm.at[idxb.at[b, 0]], gb.at[b], sems_g[b]).wait()

        def wait_scat(b):
            pltpu.make_async_copy(
                gb.at[b], acc.at[dstv.at[b]], sems_s[b]).wait()

        def compute_chunk(b):
            for i in range(B // 16):
                dstv[b, pl.ds(i * 16, 16)] = idxb[b, 1, pl.ds(i * 16, 16)]
            compute(gb.at[b], adb.at[b], iota)
            pltpu.async_copy(gb.at[b], acc.at[dstv.at[b]], sems_s[b],
                             add=True)

        _zero_and_load(gb.at[0], acc, s, sg0)
        plsc.subcore_barrier()

        issue(0, 0)
        issue(1, 1)

        def step(k, carry):
            for b in range(3):
                t = k * 3 + b

                @pl.when(t >= 1)
                def _():
                    wait_scat((b + 2) % 3)

                issue(t + 2, (b + 2) % 3)
                wait_gathers(b)
                compute_chunk(b)
            return carry

        lax.fori_loop(0, (CHUNKS - 2) // 3, step, 0)
        wait_scat(2)
        wait_gathers(0)
        compute_chunk(0)
        wait_gathers(1)
        compute_chunk(1)
        wait_scat(0)
        wait_scat(1)
        plsc.subcore_barrier()
        _write_out(acc, out_hbm, c, s, sg0)

    return body


def _make_sc_call(compute):
    mesh = plsc.VectorSubcoreMesh(core_axis_name="c", subcore_axis_name="s",
                                  num_cores=NCORE, num_subcores=NSUB)
    return pl.kernel(
        _make_sc_body(compute),
        out_type=jax.ShapeDtypeStruct((NCORE, N, ROW), jnp.float32),
        mesh=mesh,
        compiler_params=pltpu.CompilerParams(use_tc_tiling_on_sc=False,
                                             needs_layout_passes=False),
        scratch_types=[
            pltpu.VMEM_SHARED((N, ROW), jnp.float32),
            pltpu.VMEM((3, 2, B), jnp.int32),
            pltpu.VMEM((3, B, 8), jnp.float32),
            pltpu.VMEM((3, B, ROW), jnp.float32),
            pltpu.VMEM((3, B), jnp.int32),
        ] + [pltpu.SemaphoreType.DMA] * 9,
    )


@jax.jit
def kernel(x, edge_index, W1, att_src1, att_dst1, b1, W2, att_src2,
           att_dst2, b2):
    f32 = jnp.float32
    idxpk = jnp.stack([edge_index[0].reshape(E // B, B),
                       edge_index[1].reshape(E // B, B)], axis=1)

    eye8 = jnp.eye(HEADS, dtype=f32)
    asrc1_m = (att_src1[:, :, None] * eye8[:, None, :]).reshape(HEADS * HID, HEADS)
    adst1_m = (att_dst1[:, :, None] * eye8[:, None, :]).reshape(HEADS * HID, HEADS)
    e8 = jnp.repeat(eye8, HID, axis=1)
    a2_m = jnp.concatenate(
        [att_src2.reshape(D_OUT, 1), att_dst2.reshape(D_OUT, 1),
         jnp.zeros((D_OUT, 6), f32)], axis=1)

    k1 = pl.pallas_call(
        _k1_body,
        grid=(GRID,),
        in_specs=[
            pl.BlockSpec((BLK, D_IN), lambda i: (i, 0)),
            pl.BlockSpec((D_IN, HEADS * HID), lambda i: (0, 0)),
            pl.BlockSpec((HEADS * HID, HEADS), lambda i: (0, 0)),
            pl.BlockSpec((HEADS * HID, HEADS), lambda i: (0, 0)),
        ],
        out_specs=[
            pl.BlockSpec((BLK, ROW), lambda i: (i, 0)),
            pl.BlockSpec((BLK, HEADS), lambda i: (i, 0)),
        ],
        out_shape=[
            jax.ShapeDtypeStruct((N, ROW), f32),
            jax.ShapeDtypeStruct((N, HEADS), f32),
        ],
    )
    tab1, adst1 = k1(x, W1, asrc1_m, adst1_m)

    sc_l1 = _make_sc_call(_compute_l1)
    p1 = sc_l1(tab1, adst1, idxpk)

    k3 = pl.pallas_call(
        _k3_body,
        grid=(GRID,),
        in_specs=[
            pl.BlockSpec((NCORE, BLK, ROW), lambda i: (0, i, 0)),
            pl.BlockSpec((BLK, ROW), lambda i: (i, 0)),
            pl.BlockSpec((BLK, HEADS), lambda i: (i, 0)),
            pl.BlockSpec((1, HEADS * HID), lambda i: (0, 0)),
            pl.BlockSpec((HEADS * HID, D_OUT), lambda i: (0, 0)),
            pl.BlockSpec((D_OUT, HEADS), lambda i: (0, 0)),
            pl.BlockSpec((HEADS, HEADS * HID), lambda i: (0, 0)),
        ],
        out_specs=[
            pl.BlockSpec((BLK, ROW), lambda i: (i, 0)),
            pl.BlockSpec((BLK, HEADS), lambda i: (i, 0)),
        ],
        out_shape=[
            jax.ShapeDtypeStruct((N, ROW), f32),
            jax.ShapeDtypeStruct((N, HEADS), f32),
        ],
    )
    tab2, a2all = k3(p1, tab1, adst1, b1.reshape(1, -1), W2, a2_m, e8)

    sc_l2 = _make_sc_call(_compute_l2)
    p2 = sc_l2(tab2, a2all, idxpk)

    k5 = pl.pallas_call(
        _k5_body,
        grid=(GRID,),
        in_specs=[
            pl.BlockSpec((NCORE, BLK, ROW), lambda i: (0, i, 0)),
            pl.BlockSpec((BLK, ROW), lambda i: (i, 0)),
            pl.BlockSpec((BLK, HEADS), lambda i: (i, 0)),
            pl.BlockSpec((1, D_OUT), lambda i: (0, 0)),
        ],
        out_specs=pl.BlockSpec((BLK, D_OUT), lambda i: (i, 0)),
        out_shape=jax.ShapeDtypeStruct((N, D_OUT), f32),
    )
    return k5(p2, tab2, a2all, b2.reshape(1, -1))


# speedup vs baseline: 1.3097x; 1.0403x over previous
import functools

import jax
import jax.numpy as jnp
from jax import lax
from jax.experimental import pallas as pl
from jax.experimental.pallas import tpu as pltpu
from jax.experimental.pallas import tpu_sc as plsc

N = 10000
E = 320000
D_IN = 128
HID = 16
HEADS = 8
D_OUT = 128
ROW = 144

NCORE = 2
NSUB = 16
NW = NCORE * NSUB
EPW = E // NW
B = 80
CHUNKS = EPW // B
RPS = N // NSUB
BLK = 1000
GRID = N // BLK


def _k1_body(x_ref, w_ref, asrc_ref, adst_ref, tab_ref, ad_ref):
    h = jnp.dot(x_ref[...], w_ref[...], preferred_element_type=jnp.float32)
    a_s = jnp.dot(h, asrc_ref[...], preferred_element_type=jnp.float32)
    a_d = jnp.dot(h, adst_ref[...], preferred_element_type=jnp.float32)
    tab_ref[...] = jnp.concatenate(
        [h, a_s, jnp.zeros((h.shape[0], 8), jnp.float32)], axis=1)
    ad_ref[...] = a_d


def _k3_body(p_ref, tab_ref, ad_ref, b1_ref, w2_ref, a2_ref, e8_ref,
             tab2_ref, a2out_ref):
    p = p_ref[...]
    h1 = tab_ref[:, 0:128]
    a_s = tab_ref[:, 128:136]
    e = a_s + ad_ref[...]
    w = jnp.exp(jnp.where(e >= 0, e, 0.2 * e))
    wexp = jnp.dot(w, e8_ref[...], preferred_element_type=jnp.float32)
    num = p[0, :, 0:128] + p[1, :, 0:128] + h1 * wexp
    den = p[0, :, 128:136] + p[1, :, 128:136] + w
    den_exp = jnp.dot(den, e8_ref[...], preferred_element_type=jnp.float32)
    hmid = jnp.maximum(num / (den_exp + 1e-16) + b1_ref[...], 0.0)
    h2 = jnp.dot(hmid, w2_ref[...], preferred_element_type=jnp.float32)
    a2 = jnp.dot(h2, a2_ref[...], preferred_element_type=jnp.float32)
    tab2_ref[...] = jnp.concatenate(
        [h2, a2, jnp.zeros((h2.shape[0], 8), jnp.float32)], axis=1)
    a2out_ref[...] = a2


def _k5_body(p_ref, tab2_ref, a2_ref, b2_ref, out_ref):
    p = p_ref[...]
    h2 = tab2_ref[:, 0:128]
    a2 = a2_ref[...]
    e = a2[:, 0:1] + a2[:, 1:2]
    w = jnp.exp(jnp.where(e >= 0, e, 0.2 * e))
    num = p[0, :, 0:128] + p[1, :, 0:128] + h2 * w
    den = p[0, :, 128:129] + p[1, :, 128:129] + w
    out_ref[...] = num / (den + 1e-16) + b2_ref[...]


NBLK = N // B


def _zero_and_load(gbuf, acc, s, sem):
    zero = jnp.zeros((16,), jnp.float32)

    def zb(i, carry):
        row = i // (ROW // 16)
        colc = i % (ROW // 16)
        gbuf[row, pl.ds(colc * 16, 16)] = zero
        return carry

    lax.fori_loop(0, B * (ROW // 16), zb, 0)
    for t in range((NBLK + NSUB - 1) // NSUB):
        b = s + t * NSUB

        @pl.when(b < NBLK)
        def _():
            pltpu.async_copy(gbuf, acc.at[pl.ds(b * B, B)], sem)

    for t in range((NBLK + NSUB - 1) // NSUB):
        b = s + t * NSUB

        @pl.when(b < NBLK)
        def _():
            pltpu.make_async_copy(gbuf, acc.at[pl.ds(b * B, B)], sem).wait()


def _write_out(acc, out_hbm, c, s, sem):
    for t in range((NBLK + NSUB - 1) // NSUB):
        b = s + t * NSUB

        @pl.when(b < NBLK)
        def _():
            pltpu.async_copy(acc.at[pl.ds(b * B, B)],
                             out_hbm.at[c, pl.ds(b * B, B)], sem)

    for t in range((NBLK + NSUB - 1) // NSUB):
        b = s + t * NSUB

        @pl.when(b < NBLK)
        def _():
            pltpu.make_async_copy(acc.at[pl.ds(b * B, B)],
                                  out_hbm.at[c, pl.ds(b * B, B)], sem).wait()


def _compute_l1(gbuf, adbuf, iota):
    @plsc.parallel_loop(0, B // 2, 1, unroll=2)
    def _pair(j2):
        j = j2 * 2
        rows = j + (iota >> 3)
        cols = 128 + (iota & 7)
        a_s = plsc.load_gather(gbuf, [rows, cols])
        a_d = plsc.load_gather(adbuf, [rows, iota & 7])
        e = a_s + a_d
        e = jnp.where(e >= 0, e, 0.2 * e)
        w = jnp.exp(e)
        for jj in range(2):
            for k in range(HEADS):
                ws = jnp.take_along_axis(
                    w, jnp.full((16,), jj * 8 + k, jnp.int32), axis=0)
                hv = gbuf[j + jj, pl.ds(k * 16, 16)]
                gbuf[j + jj, pl.ds(k * 16, 16)] = hv * ws
        plsc.store_scatter(gbuf, [rows, cols], w)


def _compute_l2(gbuf, adbuf, iota):
    @plsc.parallel_loop(0, B // 16, 1)
    def _group(g):
        rows = g * 16 + iota
        a_s = plsc.load_gather(gbuf, [rows, jnp.broadcast_to(128, (16,))])
        a_d = plsc.load_gather(adbuf, [rows, jnp.broadcast_to(1, (16,))])
        e = a_s + a_d
        e = jnp.where(e >= 0, e, 0.2 * e)
        w16 = jnp.exp(e)
        for i in range(16):
            j = g * 16 + i
            ws = jnp.take_along_axis(
                w16, jnp.full((16,), i, jnp.int32), axis=0)
            for k in range(8):
                hv = gbuf[j, pl.ds(k * 16, 16)]
                gbuf[j, pl.ds(k * 16, 16)] = hv * ws
            gbuf[j, pl.ds(128, 16)] = jnp.where(iota == 0, ws, 0.0)


def _make_sc_body(compute):
    def body(tab_hbm, adst_hbm, idx_hbm, out_hbm,
             acc, idxb, adb, gb, dstv,
             sg0, sg1, sg2, sa0, sa1, sa2, ss0, ss1, ss2, si0, si1, si2):
        sems_g = (sg0, sg1, sg2)
        sems_a = (sa0, sa1, sa2)
        sems_s = (ss0, ss1, ss2)
        sems_i = (si0, si1, si2)
        c = lax.axis_index("c")
        s = lax.axis_index("s")
        wid = c * NSUB + s
        iota = lax.iota(jnp.int32, 16)
        rbase = wid * CHUNKS

        def issue_idx(t, b):
            pltpu.async_copy(idx_hbm.at[rbase + t], idxb.at[b], sems_i[b])

        def wait_idx(t, b):
            pltpu.make_async_copy(
                idx_hbm.at[rbase + t], idxb.at[b], sems_i[b]).wait()

        def issue_gathers(b):
            pltpu.async_copy(adst_hbm.at[idxb.at[b, 1]], adb.at[b], sems_a[b])
            pltpu.async_copy(tab_hbm.at[idxb.at[b, 0]], gb.at[b], sems_g[b])

        def wait_gathers(b):
            pltpu.make_async_copy(
                adst_hbm.at[idxb.at[b, 1]], adb.at[b], sems_a[b]).wait()
            pltpu.make_async_copy(
                tab_hbm.at[idxb.at[b, 0]], gb.at[b], sems_g[b]).wait()

        def wait_scat(b):
            pltpu.make_async_copy(
                gb.at[b], acc.at[dstv.at[b]], sems_s[b]).wait()

        def compute_chunk(b):
            for i in range(B // 16):
                dstv[b, pl.ds(i * 16, 16)] = idxb[b, 1, pl.ds(i * 16, 16)]
            compute(gb.at[b], adb.at[b], iota)
            pltpu.async_copy(gb.at[b], acc.at[dstv.at[b]], sems_s[b],
                             add=True)

        _zero_and_load(gb.at[0], acc, s, sg0)
        plsc.subcore_barrier()

        issue_idx(0, 0)
        issue_idx(1, 1)
        issue_idx(2, 2)
        wait_idx(0, 0)
        issue_gathers(0)
        wait_idx(1, 1)
        issue_gathers(1)

        def step(k, carry):
            for b in range(3):
                t = k * 3 + b
                bn = (b + 2) % 3

                @pl.when(t >= 1)
                def _():
                    wait_scat(bn)

                wait_idx(t + 2, bn)
                issue_gathers(bn)
                wait_gathers(b)

                @pl.when(t + 3 < CHUNKS)
                def _():
                    issue_idx(t + 3, b)

                compute_chunk(b)
            return carry

        lax.fori_loop(0, (CHUNKS - 2) // 3, step, 0)
        wait_scat(2)
        wait_gathers(0)
        compute_chunk(0)
        wait_gathers(1)
        compute_chunk(1)
        wait_scat(0)
        wait_scat(1)
        plsc.subcore_barrier()
        _write_out(acc, out_hbm, c, s, sg0)

    return body


def _make_sc_call(compute):
    mesh = plsc.VectorSubcoreMesh(core_axis_name="c", subcore_axis_name="s",
                                  num_cores=NCORE, num_subcores=NSUB)
    return pl.kernel(
        _make_sc_body(compute),
        out_type=jax.ShapeDtypeStruct((NCORE, N, ROW), jnp.float32),
        mesh=mesh,
        compiler_params=pltpu.CompilerParams(use_tc_tiling_on_sc=False,
                                             needs_layout_passes=False),
        scratch_types=[
            pltpu.VMEM_SHARED((N, ROW), jnp.float32),
            pltpu.VMEM((3, 2, B), jnp.int32),
            pltpu.VMEM((3, B, 8), jnp.float32),
            pltpu.VMEM((3, B, ROW), jnp.float32),
            pltpu.VMEM((3, B), jnp.int32),
        ] + [pltpu.SemaphoreType.DMA] * 12,
    )


@jax.jit
def kernel(x, edge_index, W1, att_src1, att_dst1, b1, W2, att_src2,
           att_dst2, b2):
    f32 = jnp.float32
    idxpk = jnp.stack([edge_index[0].reshape(E // B, B),
                       edge_index[1].reshape(E // B, B)], axis=1)

    eye8 = jnp.eye(HEADS, dtype=f32)
    asrc1_m = (att_src1[:, :, None] * eye8[:, None, :]).reshape(HEADS * HID, HEADS)
    adst1_m = (att_dst1[:, :, None] * eye8[:, None, :]).reshape(HEADS * HID, HEADS)
    e8 = jnp.repeat(eye8, HID, axis=1)
    a2_m = jnp.concatenate(
        [att_src2.reshape(D_OUT, 1), att_dst2.reshape(D_OUT, 1),
         jnp.zeros((D_OUT, 6), f32)], axis=1)

    k1 = pl.pallas_call(
        _k1_body,
        grid=(GRID,),
        in_specs=[
            pl.BlockSpec((BLK, D_IN), lambda i: (i, 0)),
            pl.BlockSpec((D_IN, HEADS * HID), lambda i: (0, 0)),
            pl.BlockSpec((HEADS * HID, HEADS), lambda i: (0, 0)),
            pl.BlockSpec((HEADS * HID, HEADS), lambda i: (0, 0)),
        ],
        out_specs=[
            pl.BlockSpec((BLK, ROW), lambda i: (i, 0)),
            pl.BlockSpec((BLK, HEADS), lambda i: (i, 0)),
        ],
        out_shape=[
            jax.ShapeDtypeStruct((N, ROW), f32),
            jax.ShapeDtypeStruct((N, HEADS), f32),
        ],
    )
    tab1, adst1 = k1(x, W1, asrc1_m, adst1_m)

    sc_l1 = _make_sc_call(_compute_l1)
    p1 = sc_l1(tab1, adst1, idxpk)

    k3 = pl.pallas_call(
        _k3_body,
        grid=(GRID,),
        in_specs=[
            pl.BlockSpec((NCORE, BLK, ROW), lambda i: (0, i, 0)),
            pl.BlockSpec((BLK, ROW), lambda i: (i, 0)),
            pl.BlockSpec((BLK, HEADS), lambda i: (i, 0)),
            pl.BlockSpec((1, HEADS * HID), lambda i: (0, 0)),
            pl.BlockSpec((HEADS * HID, D_OUT), lambda i: (0, 0)),
            pl.BlockSpec((D_OUT, HEADS), lambda i: (0, 0)),
            pl.BlockSpec((HEADS, HEADS * HID), lambda i: (0, 0)),
        ],
        out_specs=[
            pl.BlockSpec((BLK, ROW), lambda i: (i, 0)),
            pl.BlockSpec((BLK, HEADS), lambda i: (i, 0)),
        ],
        out_shape=[
            jax.ShapeDtypeStruct((N, ROW), f32),
            jax.ShapeDtypeStruct((N, HEADS), f32),
        ],
    )
    tab2, a2all = k3(p1, tab1, adst1, b1.reshape(1, -1), W2, a2_m, e8)

    sc_l2 = _make_sc_call(_compute_l2)
    p2 = sc_l2(tab2, a2all, idxpk)

    k5 = pl.pallas_call(
        _k5_body,
        grid=(GRID,),
        in_specs=[
            pl.BlockSpec((NCORE, BLK, ROW), lambda i: (0, i, 0)),
            pl.BlockSpec((BLK, ROW), lambda i: (i, 0)),
            pl.BlockSpec((BLK, HEADS), lambda i: (i, 0)),
            pl.BlockSpec((1, D_OUT), lambda i: (0, 0)),
        ],
        out_specs=pl.BlockSpec((BLK, D_OUT), lambda i: (i, 0)),
        out_shape=jax.ShapeDtypeStruct((N, D_OUT), f32),
    )
    return k5(p2, tab2, a2all, b2.reshape(1, -1))
